# initial kernel scaffold (unmeasured)
import jax
import jax.numpy as jnp
from jax import lax
from jax.experimental import pallas as pl
from jax.experimental.pallas import tpu as pltpu

B, S, H, Dh, Dr = 4, 256, 32, 128, 64
D = 4096
DC = 256
DC_SH = 128
BS = B * S
SCALE = float((Dh + Dr) ** -0.5)
NBLK = 512

_MESH = pl.DeviceIdType.MESH


def _gather_body(x_ref, wdkv_ref, wuk_ref, wuv_ref, wkr_ref,
                 cparts_ref, wukf_ref, wuvf_ref, kr_ref,
                 send_sems, recv_sems):
    my_x = lax.axis_index("x")
    my_y = lax.axis_index("y")
    peer = (1 - my_x, my_y)

    x2d = x_ref[...].reshape(BS, D)
    cparts_ref[my_x] = jnp.dot(x2d, wdkv_ref[...],
                               preferred_element_type=jnp.float32)
    kr_ref[...] = jnp.dot(x2d, wkr_ref[...],
                          preferred_element_type=jnp.float32)
    wukf_ref[my_x] = wuk_ref[...]
    wuvf_ref[my_x] = wuv_ref[...]

    barrier = pltpu.get_barrier_semaphore()
    pl.semaphore_signal(barrier, inc=1, device_id=peer, device_id_type=_MESH)
    pl.semaphore_wait(barrier, 1)

    rdmas = []
    for i, ref in enumerate((cparts_ref, wukf_ref, wuvf_ref)):
        r = pltpu.make_async_remote_copy(
            src_ref=ref.at[my_x],
            dst_ref=ref.at[my_x],
            send_sem=send_sems.at[i],
            recv_sem=recv_sems.at[i],
            device_id=peer,
            device_id_type=_MESH,
        )
        r.start()
        rdmas.append(r)
    for r in rdmas:
        r.wait()


def _gather(x, wdkv, wuk, wuv, wkr):
    return pl.pallas_call(
        _gather_body,
        out_shape=(
            jax.ShapeDtypeStruct((2, BS, DC_SH), jnp.float32),
            jax.ShapeDtypeStruct((2, DC_SH, D), jnp.float32),
            jax.ShapeDtypeStruct((2, DC_SH, D), jnp.float32),
            jax.ShapeDtypeStruct((BS, Dr), jnp.float32),
        ),
        in_specs=[pl.BlockSpec(memory_space=pltpu.VMEM)] * 5,
        out_specs=(
            pl.BlockSpec(memory_space=pltpu.VMEM),
            pl.BlockSpec(memory_space=pltpu.VMEM),
            pl.BlockSpec(memory_space=pltpu.VMEM),
            pl.BlockSpec(memory_space=pltpu.VMEM),
        ),
        scratch_shapes=[
            pltpu.SemaphoreType.DMA((3,)),
            pltpu.SemaphoreType.DMA((3,)),
        ],
        compiler_params=pltpu.CompilerParams(collective_id=0),
    )(x, wdkv, wuk, wuv, wkr)


def _attn_body(x_ref, c_ref, kr_ref, wq_ref, wqr_ref, wuk_ref, wuv_ref,
               o_ref):
    q = jnp.dot(x_ref[...], wq_ref[...], preferred_element_type=jnp.float32)
    qr = jnp.dot(x_ref[...], wqr_ref[...], preferred_element_type=jnp.float32)
    k = jnp.dot(c_ref[...], wuk_ref[...], preferred_element_type=jnp.float32)
    v = jnp.dot(c_ref[...], wuv_ref[...], preferred_element_type=jnp.float32)
    s = lax.dot_general(q, k, (((1,), (1,)), ((), ())),
                        preferred_element_type=jnp.float32)
    s = s + lax.dot_general(qr, kr_ref[...], (((1,), (1,)), ((), ())),
                            preferred_element_type=jnp.float32)
    s = s * SCALE
    m = jnp.max(s, axis=-1, keepdims=True)
    p = jnp.exp(s - m)
    p = p / jnp.sum(p, axis=-1, keepdims=True)
    o_ref[0] = jnp.dot(p, v, preferred_element_type=jnp.float32)


def _attn(x_b, c_b, kr_b, wq, wqr, wukf, wuvf):
    return pl.pallas_call(
        _attn_body,
        grid=(H,),
        in_specs=[
            pl.BlockSpec((S, D), lambda h: (0, 0)),
            pl.BlockSpec((S, DC), lambda h: (0, 0)),
            pl.BlockSpec((S, Dr), lambda h: (0, 0)),
            pl.BlockSpec((D, Dh), lambda h: (0, h)),
            pl.BlockSpec((D, Dr), lambda h: (0, h)),
            pl.BlockSpec((DC, Dh), lambda h: (0, h)),
            pl.BlockSpec((DC, Dh), lambda h: (0, h)),
        ],
        out_specs=pl.BlockSpec((1, S, Dh), lambda h: (h, 0, 0)),
        out_shape=jax.ShapeDtypeStruct((H, S, Dh), jnp.float32),
    )(x_b, c_b, kr_b, wq, wqr, wukf, wuvf)


def _outproj_body(o_ref, wo_ref, out_ref):
    h = pl.program_id(1)
    prod = jnp.dot(o_ref[0], wo_ref[...], preferred_element_type=jnp.float32)

    @pl.when(h == 0)
    def _():
        out_ref[...] = prod

    @pl.when(h != 0)
    def _():
        out_ref[...] += prod


def _outproj(o, wo):
    return pl.pallas_call(
        _outproj_body,
        grid=(D // NBLK, H),
        in_specs=[
            pl.BlockSpec((1, S, Dh), lambda n, h: (h, 0, 0)),
            pl.BlockSpec((Dh, NBLK), lambda n, h: (h, n)),
        ],
        out_specs=pl.BlockSpec((S, NBLK), lambda n, h: (0, n)),
        out_shape=jax.ShapeDtypeStruct((S, D), jnp.float32),
    )(o, wo)


def _allgather_body(outb_ref, out_ref, hs, send_sems, recv_sems):
    my_x = lax.axis_index("x")
    my_y = lax.axis_index("y")
    b_mine = 2 * my_x + my_y
    ypeer = (my_x, 1 - my_y)
    xpeer = (1 - my_x, my_y)

    out_ref[b_mine] = outb_ref[...]

    barrier = pltpu.get_barrier_semaphore()
    for p in (ypeer, xpeer):
        pl.semaphore_signal(barrier, inc=1, device_id=p, device_id_type=_MESH)
    pl.semaphore_wait(barrier, 2)

    pl.semaphore_signal(hs.at[0], inc=1, device_id=ypeer, device_id_type=_MESH)
    pl.semaphore_wait(hs.at[0], 1)
    r1 = pltpu.make_async_remote_copy(
        src_ref=out_ref.at[b_mine],
        dst_ref=out_ref.at[b_mine],
        send_sem=send_sems.at[0],
        recv_sem=recv_sems.at[0],
        device_id=ypeer,
        device_id_type=_MESH,
    )
    r1.start()
    r1.wait()

    pl.semaphore_signal(hs.at[1], inc=1, device_id=xpeer, device_id_type=_MESH)
    pl.semaphore_wait(hs.at[1], 1)
    row = 2 * my_x
    r2 = pltpu.make_async_remote_copy(
        src_ref=out_ref.at[pl.ds(row, 2)],
        dst_ref=out_ref.at[pl.ds(row, 2)],
        send_sem=send_sems.at[1],
        recv_sem=recv_sems.at[1],
        device_id=xpeer,
        device_id_type=_MESH,
    )
    r2.start()
    r2.wait()


def _allgather(out_b):
    return pl.pallas_call(
        _allgather_body,
        out_shape=jax.ShapeDtypeStruct((B, S, D), jnp.float32),
        in_specs=[pl.BlockSpec(memory_space=pltpu.VMEM)],
        out_specs=pl.BlockSpec(memory_space=pltpu.VMEM),
        scratch_shapes=[
            pltpu.SemaphoreType.REGULAR((2,)),
            pltpu.SemaphoreType.DMA((2,)),
            pltpu.SemaphoreType.DMA((2,)),
        ],
        compiler_params=pltpu.CompilerParams(collective_id=1),
    )(out_b)


def kernel(x, Wdkv, Wuk, Wuv, Wq, Wqr, Wkr, Wo):
    c_parts, wukf2, wuvf2, kr2d = _gather(x, Wdkv, Wuk, Wuv, Wkr)
    wukf = wukf2.reshape(DC, D)
    wuvf = wuvf2.reshape(DC, D)
    c = jnp.concatenate([c_parts[0], c_parts[1]], axis=1).reshape(B, S, DC)
    kr = kr2d.reshape(B, S, Dr)

    my_x = lax.axis_index("x")
    my_y = lax.axis_index("y")
    b = 2 * my_x + my_y
    x_b = lax.dynamic_slice_in_dim(x, b, 1, axis=0)[0]
    c_b = lax.dynamic_slice_in_dim(c, b, 1, axis=0)[0]
    kr_b = lax.dynamic_slice_in_dim(kr, b, 1, axis=0)[0]

    o = _attn(x_b, c_b, kr_b, Wq, Wqr, wukf, wuvf)
    out_b = _outproj(o, Wo)
    return _allgather(out_b)


# baseline (device time: 454369 ns/iter reference)
import jax
import jax.numpy as jnp
from jax import lax
from jax.experimental import pallas as pl
from jax.experimental.pallas import tpu as pltpu

B, S, H, Dh, Dr = 4, 256, 32, 128, 64
D = 4096
DC = 256
DC_SH = 128
BS = B * S
SCALE = float((Dh + Dr) ** -0.5)
NBLK = 512

_MESH = pl.DeviceIdType.MESH


def _gather_body(x_ref, wdkv_ref, wuk_ref, wuv_ref, wkr_ref,
                 cparts_ref, wukf_ref, wuvf_ref, kr_ref,
                 send_sems, recv_sems):
    my_x = lax.axis_index("x")
    my_y = lax.axis_index("y")
    peer = (1 - my_x, my_y)

    x2d = x_ref[...].reshape(BS, D)
    cparts_ref[my_x] = jnp.dot(x2d, wdkv_ref[...],
                               preferred_element_type=jnp.float32)
    kr_ref[...] = jnp.dot(x2d, wkr_ref[...],
                          preferred_element_type=jnp.float32)
    wukf_ref[my_x] = wuk_ref[...]
    wuvf_ref[my_x] = wuv_ref[...]

    barrier = pltpu.get_barrier_semaphore()
    pl.semaphore_signal(barrier, inc=1, device_id=peer, device_id_type=_MESH)
    pl.semaphore_wait(barrier, 1)

    rdmas = []
    for i, ref in enumerate((cparts_ref, wukf_ref, wuvf_ref)):
        r = pltpu.make_async_remote_copy(
            src_ref=ref.at[my_x],
            dst_ref=ref.at[my_x],
            send_sem=send_sems.at[i],
            recv_sem=recv_sems.at[i],
            device_id=peer,
            device_id_type=_MESH,
        )
        r.start()
        rdmas.append(r)
    for r in rdmas:
        r.wait()


def _gather(x, wdkv, wuk, wuv, wkr):
    return pl.pallas_call(
        _gather_body,
        out_shape=(
            jax.ShapeDtypeStruct((2, BS, DC_SH), jnp.float32),
            jax.ShapeDtypeStruct((2, DC_SH, D), jnp.float32),
            jax.ShapeDtypeStruct((2, DC_SH, D), jnp.float32),
            jax.ShapeDtypeStruct((BS, Dr), jnp.float32),
        ),
        in_specs=[pl.BlockSpec(memory_space=pltpu.VMEM)] * 5,
        out_specs=(
            pl.BlockSpec(memory_space=pltpu.VMEM),
            pl.BlockSpec(memory_space=pltpu.VMEM),
            pl.BlockSpec(memory_space=pltpu.VMEM),
            pl.BlockSpec(memory_space=pltpu.VMEM),
        ),
        scratch_shapes=[
            pltpu.SemaphoreType.DMA((3,)),
            pltpu.SemaphoreType.DMA((3,)),
        ],
        compiler_params=pltpu.CompilerParams(collective_id=0),
    )(x, wdkv, wuk, wuv, wkr)


def _attn_body(x_ref, c_ref, kr_ref, wq_ref, wqr_ref, wuk_ref, wuv_ref,
               o_ref):
    h = pl.program_id(0)
    q = jnp.dot(x_ref[...], wq_ref[...], preferred_element_type=jnp.float32)
    qr2 = jnp.dot(x_ref[...], wqr_ref[...], preferred_element_type=jnp.float32)
    lane = lax.broadcasted_iota(jnp.int32, (S, 2 * Dr), 1)
    qr2 = jnp.where((lane // Dr) == (h % 2), qr2, 0.0)
    k = jnp.dot(c_ref[...], wuk_ref[...], preferred_element_type=jnp.float32)
    v = jnp.dot(c_ref[...], wuv_ref[...], preferred_element_type=jnp.float32)
    s = lax.dot_general(q, k, (((1,), (1,)), ((), ())),
                        preferred_element_type=jnp.float32)
    kr2 = jnp.concatenate([kr_ref[...], kr_ref[...]], axis=1)
    s = s + lax.dot_general(qr2, kr2, (((1,), (1,)), ((), ())),
                            preferred_element_type=jnp.float32)
    s = s * SCALE
    m = jnp.max(s, axis=-1, keepdims=True)
    p = jnp.exp(s - m)
    p = p / jnp.sum(p, axis=-1, keepdims=True)
    o_ref[0] = jnp.dot(p, v, preferred_element_type=jnp.float32)


def _attn(x_b, c_b, kr_b, wq, wqr, wukf, wuvf):
    return pl.pallas_call(
        _attn_body,
        grid=(H,),
        in_specs=[
            pl.BlockSpec((S, D), lambda h: (0, 0)),
            pl.BlockSpec((S, DC), lambda h: (0, 0)),
            pl.BlockSpec((S, Dr), lambda h: (0, 0)),
            pl.BlockSpec((D, Dh), lambda h: (0, h)),
            pl.BlockSpec((D, 2 * Dr), lambda h: (0, h // 2)),
            pl.BlockSpec((DC, Dh), lambda h: (0, h)),
            pl.BlockSpec((DC, Dh), lambda h: (0, h)),
        ],
        out_specs=pl.BlockSpec((1, S, Dh), lambda h: (h, 0, 0)),
        out_shape=jax.ShapeDtypeStruct((H, S, Dh), jnp.float32),
    )(x_b, c_b, kr_b, wq, wqr, wukf, wuvf)


def _outproj_body(o_ref, wo_ref, out_ref):
    h = pl.program_id(1)
    prod = jnp.dot(o_ref[0], wo_ref[...], preferred_element_type=jnp.float32)

    @pl.when(h == 0)
    def _():
        out_ref[...] = prod

    @pl.when(h != 0)
    def _():
        out_ref[...] += prod


def _outproj(o, wo):
    return pl.pallas_call(
        _outproj_body,
        grid=(D // NBLK, H),
        in_specs=[
            pl.BlockSpec((1, S, Dh), lambda n, h: (h, 0, 0)),
            pl.BlockSpec((Dh, NBLK), lambda n, h: (h, n)),
        ],
        out_specs=pl.BlockSpec((S, NBLK), lambda n, h: (0, n)),
        out_shape=jax.ShapeDtypeStruct((S, D), jnp.float32),
    )(o, wo)


def _allgather_body(outb_ref, out_ref, hs, send_sems, recv_sems):
    my_x = lax.axis_index("x")
    my_y = lax.axis_index("y")
    b_mine = 2 * my_x + my_y
    ypeer = (my_x, 1 - my_y)
    xpeer = (1 - my_x, my_y)

    out_ref[b_mine] = outb_ref[...]

    barrier = pltpu.get_barrier_semaphore()
    for p in (ypeer, xpeer):
        pl.semaphore_signal(barrier, inc=1, device_id=p, device_id_type=_MESH)
    pl.semaphore_wait(barrier, 2)

    pl.semaphore_signal(hs.at[0], inc=1, device_id=ypeer, device_id_type=_MESH)
    pl.semaphore_wait(hs.at[0], 1)
    r1 = pltpu.make_async_remote_copy(
        src_ref=out_ref.at[b_mine],
        dst_ref=out_ref.at[b_mine],
        send_sem=send_sems.at[0],
        recv_sem=recv_sems.at[0],
        device_id=ypeer,
        device_id_type=_MESH,
    )
    r1.start()
    r1.wait()

    pl.semaphore_signal(hs.at[1], inc=1, device_id=xpeer, device_id_type=_MESH)
    pl.semaphore_wait(hs.at[1], 1)
    row = 2 * my_x
    r2 = pltpu.make_async_remote_copy(
        src_ref=out_ref.at[pl.ds(row, 2)],
        dst_ref=out_ref.at[pl.ds(row, 2)],
        send_sem=send_sems.at[1],
        recv_sem=recv_sems.at[1],
        device_id=xpeer,
        device_id_type=_MESH,
    )
    r2.start()
    r2.wait()


def _allgather(out_b):
    return pl.pallas_call(
        _allgather_body,
        out_shape=jax.ShapeDtypeStruct((B, S, D), jnp.float32),
        in_specs=[pl.BlockSpec(memory_space=pltpu.VMEM)],
        out_specs=pl.BlockSpec(memory_space=pltpu.VMEM),
        scratch_shapes=[
            pltpu.SemaphoreType.REGULAR((2,)),
            pltpu.SemaphoreType.DMA((2,)),
            pltpu.SemaphoreType.DMA((2,)),
        ],
        compiler_params=pltpu.CompilerParams(collective_id=1),
    )(out_b)


def kernel(x, Wdkv, Wuk, Wuv, Wq, Wqr, Wkr, Wo):
    c_parts, wukf2, wuvf2, kr2d = _gather(x, Wdkv, Wuk, Wuv, Wkr)
    wukf = wukf2.reshape(DC, D)
    wuvf = wuvf2.reshape(DC, D)
    c = jnp.concatenate([c_parts[0], c_parts[1]], axis=1).reshape(B, S, DC)
    kr = kr2d.reshape(B, S, Dr)

    my_x = lax.axis_index("x")
    my_y = lax.axis_index("y")
    b = 2 * my_x + my_y
    x_b = lax.dynamic_slice_in_dim(x, b, 1, axis=0)[0]
    c_b = lax.dynamic_slice_in_dim(c, b, 1, axis=0)[0]
    kr_b = lax.dynamic_slice_in_dim(kr, b, 1, axis=0)[0]

    o = _attn(x_b, c_b, kr_b, Wq, Wqr, wukf, wuvf)
    out_b = _outproj(o, Wo)
    return _allgather(out_b)


# device time: 241881 ns/iter; 1.8785x vs baseline; 1.8785x over previous
import jax
import jax.numpy as jnp
from jax import lax
from jax.experimental import pallas as pl
from jax.experimental.pallas import tpu as pltpu

B, S, H, Dh, Dr = 4, 256, 32, 128, 64
D = 4096
DC = 256
DC_SH = 128
BS = B * S
SCALE = float((Dh + Dr) ** -0.5)
NBLK = 512

_MESH = pl.DeviceIdType.MESH


def _gather_body(x_ref, wdkv_ref, wuk_ref, wuv_ref, wkr_ref,
                 cparts_ref, wukf_ref, wuvf_ref, kr_ref,
                 send_sems, recv_sems):
    my_x = lax.axis_index("x")
    my_y = lax.axis_index("y")
    peer = (1 - my_x, my_y)

    x2d = x_ref[...].reshape(BS, D)
    cparts_ref[my_x] = jnp.dot(x2d, wdkv_ref[...],
                               preferred_element_type=jnp.float32)
    kr_ref[...] = jnp.dot(x2d, wkr_ref[...],
                          preferred_element_type=jnp.float32)
    wukf_ref[my_x] = wuk_ref[...]
    wuvf_ref[my_x] = wuv_ref[...]

    barrier = pltpu.get_barrier_semaphore()
    pl.semaphore_signal(barrier, inc=1, device_id=peer, device_id_type=_MESH)
    pl.semaphore_wait(barrier, 1)

    rdmas = []
    for i, ref in enumerate((cparts_ref, wukf_ref, wuvf_ref)):
        r = pltpu.make_async_remote_copy(
            src_ref=ref.at[my_x],
            dst_ref=ref.at[my_x],
            send_sem=send_sems.at[i],
            recv_sem=recv_sems.at[i],
            device_id=peer,
            device_id_type=_MESH,
        )
        r.start()
        rdmas.append(r)
    for r in rdmas:
        r.wait()


def _gather(x, wdkv, wuk, wuv, wkr):
    return pl.pallas_call(
        _gather_body,
        out_shape=(
            jax.ShapeDtypeStruct((2, BS, DC_SH), jnp.float32),
            jax.ShapeDtypeStruct((2, DC_SH, D), jnp.float32),
            jax.ShapeDtypeStruct((2, DC_SH, D), jnp.float32),
            jax.ShapeDtypeStruct((BS, Dr), jnp.float32),
        ),
        in_specs=[pl.BlockSpec(memory_space=pltpu.VMEM)] * 5,
        out_specs=(
            pl.BlockSpec(memory_space=pltpu.VMEM),
            pl.BlockSpec(memory_space=pltpu.VMEM),
            pl.BlockSpec(memory_space=pltpu.VMEM),
            pl.BlockSpec(memory_space=pltpu.VMEM),
        ),
        scratch_shapes=[
            pltpu.SemaphoreType.DMA((3,)),
            pltpu.SemaphoreType.DMA((3,)),
        ],
        compiler_params=pltpu.CompilerParams(collective_id=0),
    )(x, wdkv, wuk, wuv, wkr)


def _attn_body(x_ref, c_ref, kr_ref, wq_ref, wqr_ref, wuk_ref, wuv_ref,
               o_ref):
    h = pl.program_id(0)
    q = jnp.dot(x_ref[...], wq_ref[...], preferred_element_type=jnp.float32)
    qr2 = jnp.dot(x_ref[...], wqr_ref[...], preferred_element_type=jnp.float32)
    lane = lax.broadcasted_iota(jnp.int32, (S, 2 * Dr), 1)
    qr2 = jnp.where((lane // Dr) == (h % 2), qr2, 0.0)
    k = jnp.dot(c_ref[...], wuk_ref[...], preferred_element_type=jnp.float32)
    v = jnp.dot(c_ref[...], wuv_ref[...], preferred_element_type=jnp.float32)
    s = lax.dot_general(q, k, (((1,), (1,)), ((), ())),
                        preferred_element_type=jnp.float32)
    kr2 = jnp.concatenate([kr_ref[...], kr_ref[...]], axis=1)
    s = s + lax.dot_general(qr2, kr2, (((1,), (1,)), ((), ())),
                            preferred_element_type=jnp.float32)
    s = s * SCALE
    m = jnp.max(s, axis=-1, keepdims=True)
    p = jnp.exp(s - m)
    p = p / jnp.sum(p, axis=-1, keepdims=True)
    o_ref[...] = jnp.dot(p, v, preferred_element_type=jnp.float32)


def _attn(x_b, c_b, kr_b, wq, wqr, wukf, wuvf):
    return pl.pallas_call(
        _attn_body,
        grid=(H,),
        in_specs=[
            pl.BlockSpec((S, D), lambda h: (0, 0)),
            pl.BlockSpec((S, DC), lambda h: (0, 0)),
            pl.BlockSpec((S, Dr), lambda h: (0, 0)),
            pl.BlockSpec((D, Dh), lambda h: (0, h)),
            pl.BlockSpec((D, 2 * Dr), lambda h: (0, h // 2)),
            pl.BlockSpec((DC, Dh), lambda h: (0, h)),
            pl.BlockSpec((DC, Dh), lambda h: (0, h)),
        ],
        out_specs=pl.BlockSpec((S, Dh), lambda h: (0, h)),
        out_shape=jax.ShapeDtypeStruct((S, H * Dh), jnp.float32),
    )(x_b, c_b, kr_b, wq, wqr, wukf, wuvf)


def _outproj_body(o_ref, wo_ref, out_ref):
    out_ref[...] = jnp.dot(o_ref[...], wo_ref[...],
                           preferred_element_type=jnp.float32)


def _outproj(o, wo):
    return pl.pallas_call(
        _outproj_body,
        grid=(D // NBLK,),
        in_specs=[
            pl.BlockSpec((S, H * Dh), lambda n: (0, 0)),
            pl.BlockSpec((H * Dh, NBLK), lambda n: (0, n)),
        ],
        out_specs=pl.BlockSpec((S, NBLK), lambda n: (0, n)),
        out_shape=jax.ShapeDtypeStruct((S, D), jnp.float32),
    )(o, wo)


def _allgather_body(outb_ref, out_ref, hs, send_sems, recv_sems):
    my_x = lax.axis_index("x")
    my_y = lax.axis_index("y")
    b_mine = 2 * my_x + my_y
    b_y = 2 * my_x + (1 - my_y)
    b_x = 2 * (1 - my_x) + my_y
    ypeer = (my_x, 1 - my_y)
    xpeer = (1 - my_x, my_y)

    out_ref[b_mine] = outb_ref[...]

    barrier = pltpu.get_barrier_semaphore()
    for p in (ypeer, xpeer):
        pl.semaphore_signal(barrier, inc=1, device_id=p, device_id_type=_MESH)
    pl.semaphore_wait(barrier, 2)
    pl.semaphore_signal(hs.at[0], inc=1, device_id=ypeer, device_id_type=_MESH)
    pl.semaphore_signal(hs.at[1], inc=1, device_id=xpeer, device_id_type=_MESH)
    pl.semaphore_wait(hs.at[0], 1)
    pl.semaphore_wait(hs.at[1], 1)

    r1y = pltpu.make_async_remote_copy(
        src_ref=out_ref.at[b_mine], dst_ref=out_ref.at[b_mine],
        send_sem=send_sems.at[0], recv_sem=recv_sems.at[0],
        device_id=ypeer, device_id_type=_MESH,
    )
    r1x = pltpu.make_async_remote_copy(
        src_ref=out_ref.at[b_mine], dst_ref=out_ref.at[b_mine],
        send_sem=send_sems.at[1], recv_sem=recv_sems.at[1],
        device_id=xpeer, device_id_type=_MESH,
    )
    r1y.start()
    r1x.start()
    r1y.wait()
    r1x.wait()

    r2y = pltpu.make_async_remote_copy(
        src_ref=out_ref.at[b_x, :, pl.ds(0, D // 2)],
        dst_ref=out_ref.at[b_x, :, pl.ds(0, D // 2)],
        send_sem=send_sems.at[2], recv_sem=recv_sems.at[2],
        device_id=ypeer, device_id_type=_MESH,
    )
    r2x = pltpu.make_async_remote_copy(
        src_ref=out_ref.at[b_y, :, pl.ds(D // 2, D // 2)],
        dst_ref=out_ref.at[b_y, :, pl.ds(D // 2, D // 2)],
        send_sem=send_sems.at[3], recv_sem=recv_sems.at[3],
        device_id=xpeer, device_id_type=_MESH,
    )
    r2y.start()
    r2x.start()
    r2y.wait()
    r2x.wait()


def _allgather(out_b):
    return pl.pallas_call(
        _allgather_body,
        out_shape=jax.ShapeDtypeStruct((B, S, D), jnp.float32),
        in_specs=[pl.BlockSpec(memory_space=pltpu.VMEM)],
        out_specs=pl.BlockSpec(memory_space=pltpu.VMEM),
        scratch_shapes=[
            pltpu.SemaphoreType.REGULAR((2,)),
            pltpu.SemaphoreType.DMA((4,)),
            pltpu.SemaphoreType.DMA((4,)),
        ],
        compiler_params=pltpu.CompilerParams(collective_id=1),
    )(out_b)


def kernel(x, Wdkv, Wuk, Wuv, Wq, Wqr, Wkr, Wo):
    c_parts, wukf2, wuvf2, kr2d = _gather(x, Wdkv, Wuk, Wuv, Wkr)
    wukf = wukf2.reshape(DC, D)
    wuvf = wuvf2.reshape(DC, D)
    c = jnp.concatenate([c_parts[0], c_parts[1]], axis=1).reshape(B, S, DC)
    kr = kr2d.reshape(B, S, Dr)

    my_x = lax.axis_index("x")
    my_y = lax.axis_index("y")
    b = 2 * my_x + my_y
    x_b = lax.dynamic_slice_in_dim(x, b, 1, axis=0)[0]
    c_b = lax.dynamic_slice_in_dim(c, b, 1, axis=0)[0]
    kr_b = lax.dynamic_slice_in_dim(kr, b, 1, axis=0)[0]

    o = _attn(x_b, c_b, kr_b, Wq, Wqr, wukf, wuvf)
    out_b = _outproj(o, Wo)
    return _allgather(out_b)


# device time: 203867 ns/iter; 2.2288x vs baseline; 1.1865x over previous
import jax
import jax.numpy as jnp
from jax import lax
from jax.experimental import pallas as pl
from jax.experimental.pallas import tpu as pltpu

B, S, H, Dh, Dr = 4, 256, 32, 128, 64
D = 4096
DC = 256
DC_SH = 128
SCALE = float((Dh + Dr) ** -0.5)
NBLK = 512

_MESH = pl.DeviceIdType.MESH


def _ab_body(x_b_ref, x_bp_ref, wdkv_ref, wuk_ref, wuv_ref, wkr_ref,
             wq_ref, wqr_ref, o_ref,
             q_scr, qr_scr, kr_scr, c_scr, cpeer_scr, wukf, wuvf,
             send_sems, recv_sems):
    p = pl.program_id(0)
    my_x = lax.axis_index("x")
    my_y = lax.axis_index("y")
    xpeer = (1 - my_x, my_y)

    my_cols = pl.ds(my_x * DC_SH, DC_SH)
    peer_cols = pl.ds((1 - my_x) * DC_SH, DC_SH)

    @pl.when(p == 0)
    def _():
        c_scr[:, my_cols] = jnp.dot(x_b_ref[...], wdkv_ref[...],
                                    preferred_element_type=jnp.float32)
        cpeer_scr[...] = jnp.dot(x_bp_ref[...], wdkv_ref[...],
                                 preferred_element_type=jnp.float32)
        kr_scr[...] = jnp.dot(x_b_ref[...], wkr_ref[...],
                              preferred_element_type=jnp.float32)
        wukf[my_cols, :] = wuk_ref[...]
        wuvf[my_cols, :] = wuv_ref[...]

        barrier = pltpu.get_barrier_semaphore()
        pl.semaphore_signal(barrier, inc=1, device_id=xpeer,
                            device_id_type=_MESH)
        pl.semaphore_wait(barrier, 1)
        pltpu.make_async_remote_copy(
            src_ref=cpeer_scr.at[...],
            dst_ref=c_scr.at[:, my_cols],
            send_sem=send_sems.at[0], recv_sem=recv_sems.at[0],
            device_id=xpeer, device_id_type=_MESH,
        ).start()
        pltpu.make_async_remote_copy(
            src_ref=wukf.at[my_cols, :],
            dst_ref=wukf.at[my_cols, :],
            send_sem=send_sems.at[1], recv_sem=recv_sems.at[1],
            device_id=xpeer, device_id_type=_MESH,
        ).start()
        pltpu.make_async_remote_copy(
            src_ref=wuvf.at[my_cols, :],
            dst_ref=wuvf.at[my_cols, :],
            send_sem=send_sems.at[2], recv_sem=recv_sems.at[2],
            device_id=xpeer, device_id_type=_MESH,
        ).start()

    @pl.when(p < H)
    def _():
        q_scr[:, pl.ds(p * Dh, Dh)] = jnp.dot(
            x_b_ref[...], wq_ref[...], preferred_element_type=jnp.float32)

        @pl.when(p % 2 == 0)
        def _():
            qr_scr[p // 2] = jnp.dot(x_b_ref[...], wqr_ref[...],
                                     preferred_element_type=jnp.float32)

    @pl.when(p == H)
    def _():
        pltpu.make_async_remote_copy(
            src_ref=cpeer_scr.at[...], dst_ref=c_scr.at[:, peer_cols],
            send_sem=send_sems.at[0], recv_sem=recv_sems.at[0],
            device_id=xpeer, device_id_type=_MESH,
        ).wait_recv()
        pltpu.make_async_remote_copy(
            src_ref=wukf.at[my_cols, :], dst_ref=wukf.at[peer_cols, :],
            send_sem=send_sems.at[1], recv_sem=recv_sems.at[1],
            device_id=xpeer, device_id_type=_MESH,
        ).wait_recv()
        pltpu.make_async_remote_copy(
            src_ref=wuvf.at[my_cols, :], dst_ref=wuvf.at[peer_cols, :],
            send_sem=send_sems.at[2], recv_sem=recv_sems.at[2],
            device_id=xpeer, device_id_type=_MESH,
        ).wait_recv()
        pltpu.make_async_remote_copy(
            src_ref=cpeer_scr.at[...], dst_ref=c_scr.at[:, my_cols],
            send_sem=send_sems.at[0], recv_sem=recv_sems.at[0],
            device_id=xpeer, device_id_type=_MESH,
        ).wait_send()
        pltpu.make_async_remote_copy(
            src_ref=wukf.at[my_cols, :], dst_ref=wukf.at[my_cols, :],
            send_sem=send_sems.at[1], recv_sem=recv_sems.at[1],
            device_id=xpeer, device_id_type=_MESH,
        ).wait_send()
        pltpu.make_async_remote_copy(
            src_ref=wuvf.at[my_cols, :], dst_ref=wuvf.at[my_cols, :],
            send_sem=send_sems.at[2], recv_sem=recv_sems.at[2],
            device_id=xpeer, device_id_type=_MESH,
        ).wait_send()

    @pl.when(p >= H)
    def _():
        h = p - H
        head_cols = pl.ds(h * Dh, Dh)
        k = jnp.dot(c_scr[...], wukf[:, head_cols],
                    preferred_element_type=jnp.float32)
        v = jnp.dot(c_scr[...], wuvf[:, head_cols],
                    preferred_element_type=jnp.float32)
        q = q_scr[:, head_cols]
        s = lax.dot_general(q, k, (((1,), (1,)), ((), ())),
                            preferred_element_type=jnp.float32)
        qr2 = qr_scr[h // 2]
        lane = lax.broadcasted_iota(jnp.int32, (S, 2 * Dr), 1)
        qr2 = jnp.where((lane // Dr) == (h % 2), qr2, 0.0)
        kr2 = jnp.concatenate([kr_scr[...], kr_scr[...]], axis=1)
        s = s + lax.dot_general(qr2, kr2, (((1,), (1,)), ((), ())),
                                preferred_element_type=jnp.float32)
        s = s * SCALE
        m = jnp.max(s, axis=-1, keepdims=True)
        pr = jnp.exp(s - m)
        pr = pr / jnp.sum(pr, axis=-1, keepdims=True)
        o_ref[...] = jnp.dot(pr, v, preferred_element_type=jnp.float32)


def _ab(x_b, x_bp, wdkv, wuk, wuv, wkr, wq, wqr):
    return pl.pallas_call(
        _ab_body,
        grid=(2 * H,),
        in_specs=[
            pl.BlockSpec((S, D), lambda p: (0, 0)),
            pl.BlockSpec((S, D), lambda p: (0, 0)),
            pl.BlockSpec((D, DC_SH), lambda p: (0, 0)),
            pl.BlockSpec((DC_SH, D), lambda p: (0, 0)),
            pl.BlockSpec((DC_SH, D), lambda p: (0, 0)),
            pl.BlockSpec((D, Dr), lambda p: (0, 0)),
            pl.BlockSpec((D, Dh), lambda p: (0, jnp.minimum(p, H - 1))),
            pl.BlockSpec((D, 2 * Dr), lambda p: (0, jnp.minimum(p, H - 1) // 2)),
        ],
        out_specs=pl.BlockSpec(
            (S, Dh), lambda p: (0, jnp.clip(p - H, 0, H - 1))),
        out_shape=jax.ShapeDtypeStruct((S, H * Dh), jnp.float32),
        scratch_shapes=[
            pltpu.VMEM((S, H * Dh), jnp.float32),
            pltpu.VMEM((H // 2, S, 2 * Dr), jnp.float32),
            pltpu.VMEM((S, Dr), jnp.float32),
            pltpu.VMEM((S, DC), jnp.float32),
            pltpu.VMEM((S, DC_SH), jnp.float32),
            pltpu.VMEM((DC, D), jnp.float32),
            pltpu.VMEM((DC, D), jnp.float32),
            pltpu.SemaphoreType.DMA((3,)),
            pltpu.SemaphoreType.DMA((3,)),
        ],
        compiler_params=pltpu.CompilerParams(collective_id=0),
    )(x_b, x_bp, wdkv, wuk, wuv, wkr, wq, wqr)


def _outproj_body(o_ref, wo_ref, out_ref):
    out_ref[...] = jnp.dot(o_ref[...], wo_ref[...],
                           preferred_element_type=jnp.float32)


def _outproj(o, wo):
    return pl.pallas_call(
        _outproj_body,
        grid=(D // NBLK,),
        in_specs=[
            pl.BlockSpec((S, H * Dh), lambda n: (0, 0)),
            pl.BlockSpec((H * Dh, NBLK), lambda n: (0, n)),
        ],
        out_specs=pl.BlockSpec((S, NBLK), lambda n: (0, n)),
        out_shape=jax.ShapeDtypeStruct((S, D), jnp.float32),
    )(o, wo)


def _allgather_body(outb_ref, out_ref, hs, send_sems, recv_sems):
    my_x = lax.axis_index("x")
    my_y = lax.axis_index("y")
    b_mine = 2 * my_x + my_y
    b_y = 2 * my_x + (1 - my_y)
    b_x = 2 * (1 - my_x) + my_y
    ypeer = (my_x, 1 - my_y)
    xpeer = (1 - my_x, my_y)

    out_ref[b_mine] = outb_ref[...]

    barrier = pltpu.get_barrier_semaphore()
    for p in (ypeer, xpeer):
        pl.semaphore_signal(barrier, inc=1, device_id=p, device_id_type=_MESH)
    pl.semaphore_wait(barrier, 2)
    pl.semaphore_signal(hs.at[0], inc=1, device_id=ypeer, device_id_type=_MESH)
    pl.semaphore_signal(hs.at[1], inc=1, device_id=xpeer, device_id_type=_MESH)
    pl.semaphore_wait(hs.at[0], 1)
    pl.semaphore_wait(hs.at[1], 1)

    r1y = pltpu.make_async_remote_copy(
        src_ref=out_ref.at[b_mine], dst_ref=out_ref.at[b_mine],
        send_sem=send_sems.at[0], recv_sem=recv_sems.at[0],
        device_id=ypeer, device_id_type=_MESH,
    )
    r1x = pltpu.make_async_remote_copy(
        src_ref=out_ref.at[b_mine], dst_ref=out_ref.at[b_mine],
        send_sem=send_sems.at[1], recv_sem=recv_sems.at[1],
        device_id=xpeer, device_id_type=_MESH,
    )
    r1y.start()
    r1x.start()
    r1y.wait()
    r1x.wait()

    r2y = pltpu.make_async_remote_copy(
        src_ref=out_ref.at[b_x, :, pl.ds(0, D // 2)],
        dst_ref=out_ref.at[b_x, :, pl.ds(0, D // 2)],
        send_sem=send_sems.at[2], recv_sem=recv_sems.at[2],
        device_id=ypeer, device_id_type=_MESH,
    )
    r2x = pltpu.make_async_remote_copy(
        src_ref=out_ref.at[b_y, :, pl.ds(D // 2, D // 2)],
        dst_ref=out_ref.at[b_y, :, pl.ds(D // 2, D // 2)],
        send_sem=send_sems.at[3], recv_sem=recv_sems.at[3],
        device_id=xpeer, device_id_type=_MESH,
    )
    r2y.start()
    r2x.start()
    r2y.wait()
    r2x.wait()


def _allgather(out_b):
    return pl.pallas_call(
        _allgather_body,
        out_shape=jax.ShapeDtypeStruct((B, S, D), jnp.float32),
        in_specs=[pl.BlockSpec(memory_space=pltpu.VMEM)],
        out_specs=pl.BlockSpec(memory_space=pltpu.VMEM),
        scratch_shapes=[
            pltpu.SemaphoreType.REGULAR((2,)),
            pltpu.SemaphoreType.DMA((4,)),
            pltpu.SemaphoreType.DMA((4,)),
        ],
        compiler_params=pltpu.CompilerParams(collective_id=1),
    )(out_b)


def kernel(x, Wdkv, Wuk, Wuv, Wq, Wqr, Wkr, Wo):
    my_x = lax.axis_index("x")
    my_y = lax.axis_index("y")
    b_mine = 2 * my_x + my_y
    b_xpeer = 2 * (1 - my_x) + my_y
    x_b = lax.dynamic_slice_in_dim(x, b_mine, 1, axis=0)[0]
    x_bp = lax.dynamic_slice_in_dim(x, b_xpeer, 1, axis=0)[0]

    o = _ab(x_b, x_bp, Wdkv, Wuk, Wuv, Wkr, Wq, Wqr)
    out_b = _outproj(o, Wo)
    return _allgather(out_b)


# device time: 185791 ns/iter; 2.4456x vs baseline; 1.0973x over previous
import jax
import jax.numpy as jnp
from jax import lax
from jax.experimental import pallas as pl
from jax.experimental.pallas import tpu as pltpu

B, S, H, Dh, Dr = 4, 256, 32, 128, 64
D = 4096
DC = 256
DC_SH = 128
SCALE = float((Dh + Dr) ** -0.5)
NBLK = 512

_MESH = pl.DeviceIdType.MESH


def _ab_body(x_b_ref, x_bp_ref, wdkv_ref, wuk_ref, wuv_ref, wkr_ref,
             wq_ref, wqr_ref, o_ref,
             q_scr, qr_scr, kr_scr, c_scr, cpeer_scr, wukf, wuvf,
             send_sems, recv_sems):
    p = pl.program_id(0)
    my_x = lax.axis_index("x")
    my_y = lax.axis_index("y")
    xpeer = (1 - my_x, my_y)

    my_cols = pl.ds(my_x * DC_SH, DC_SH)
    peer_cols = pl.ds((1 - my_x) * DC_SH, DC_SH)

    @pl.when(p == 0)
    def _():
        c_scr[:, my_cols] = jnp.dot(x_b_ref[...], wdkv_ref[...],
                                    preferred_element_type=jnp.float32)
        cpeer_scr[...] = jnp.dot(x_bp_ref[...], wdkv_ref[...],
                                 preferred_element_type=jnp.float32)
        kr_scr[...] = jnp.dot(x_b_ref[...], wkr_ref[...],
                              preferred_element_type=jnp.float32)
        wukf[my_cols, :] = wuk_ref[...]
        wuvf[my_cols, :] = wuv_ref[...]

        barrier = pltpu.get_barrier_semaphore()
        pl.semaphore_signal(barrier, inc=1, device_id=xpeer,
                            device_id_type=_MESH)
        pl.semaphore_wait(barrier, 1)
        pltpu.make_async_remote_copy(
            src_ref=cpeer_scr.at[...],
            dst_ref=c_scr.at[:, my_cols],
            send_sem=send_sems.at[0], recv_sem=recv_sems.at[0],
            device_id=xpeer, device_id_type=_MESH,
        ).start()
        pltpu.make_async_remote_copy(
            src_ref=wukf.at[my_cols, :],
            dst_ref=wukf.at[my_cols, :],
            send_sem=send_sems.at[1], recv_sem=recv_sems.at[1],
            device_id=xpeer, device_id_type=_MESH,
        ).start()
        pltpu.make_async_remote_copy(
            src_ref=wuvf.at[my_cols, :],
            dst_ref=wuvf.at[my_cols, :],
            send_sem=send_sems.at[2], recv_sem=recv_sems.at[2],
            device_id=xpeer, device_id_type=_MESH,
        ).start()

    @pl.when(p < H)
    def _():
        q_scr[:, pl.ds(p * Dh, Dh)] = jnp.dot(
            x_b_ref[...], wq_ref[...], preferred_element_type=jnp.float32)

        @pl.when(p % 2 == 0)
        def _():
            qr_scr[p // 2] = jnp.dot(x_b_ref[...], wqr_ref[...],
                                     preferred_element_type=jnp.float32)

    @pl.when(p == H)
    def _():
        pltpu.make_async_remote_copy(
            src_ref=cpeer_scr.at[...], dst_ref=c_scr.at[:, peer_cols],
            send_sem=send_sems.at[0], recv_sem=recv_sems.at[0],
            device_id=xpeer, device_id_type=_MESH,
        ).wait_recv()
        pltpu.make_async_remote_copy(
            src_ref=wukf.at[my_cols, :], dst_ref=wukf.at[peer_cols, :],
            send_sem=send_sems.at[1], recv_sem=recv_sems.at[1],
            device_id=xpeer, device_id_type=_MESH,
        ).wait_recv()
        pltpu.make_async_remote_copy(
            src_ref=wuvf.at[my_cols, :], dst_ref=wuvf.at[peer_cols, :],
            send_sem=send_sems.at[2], recv_sem=recv_sems.at[2],
            device_id=xpeer, device_id_type=_MESH,
        ).wait_recv()
        pltpu.make_async_remote_copy(
            src_ref=cpeer_scr.at[...], dst_ref=c_scr.at[:, my_cols],
            send_sem=send_sems.at[0], recv_sem=recv_sems.at[0],
            device_id=xpeer, device_id_type=_MESH,
        ).wait_send()
        pltpu.make_async_remote_copy(
            src_ref=wukf.at[my_cols, :], dst_ref=wukf.at[my_cols, :],
            send_sem=send_sems.at[1], recv_sem=recv_sems.at[1],
            device_id=xpeer, device_id_type=_MESH,
        ).wait_send()
        pltpu.make_async_remote_copy(
            src_ref=wuvf.at[my_cols, :], dst_ref=wuvf.at[my_cols, :],
            send_sem=send_sems.at[2], recv_sem=recv_sems.at[2],
            device_id=xpeer, device_id_type=_MESH,
        ).wait_send()

    @pl.when(p >= H)
    def _():
        h = p - H
        head_cols = pl.ds(h * Dh, Dh)
        k = jnp.dot(c_scr[...], wukf[:, head_cols],
                    preferred_element_type=jnp.float32)
        v = jnp.dot(c_scr[...], wuvf[:, head_cols],
                    preferred_element_type=jnp.float32)
        q = q_scr[:, head_cols]
        s = lax.dot_general(q, k, (((1,), (1,)), ((), ())),
                            preferred_element_type=jnp.float32)
        qr2 = qr_scr[h // 2]
        lane = lax.broadcasted_iota(jnp.int32, (S, 2 * Dr), 1)
        qr2 = jnp.where((lane // Dr) == (h % 2), qr2, 0.0)
        kr2 = jnp.concatenate([kr_scr[...], kr_scr[...]], axis=1)
        s = s + lax.dot_general(qr2, kr2, (((1,), (1,)), ((), ())),
                                preferred_element_type=jnp.float32)
        s = s * SCALE
        m = jnp.max(s, axis=-1, keepdims=True)
        pr = jnp.exp(s - m)
        pr = pr / jnp.sum(pr, axis=-1, keepdims=True)
        o_ref[...] = jnp.dot(pr, v, preferred_element_type=jnp.float32)


def _ab(x_b, x_bp, wdkv, wuk, wuv, wkr, wq, wqr):
    return pl.pallas_call(
        _ab_body,
        grid=(2 * H,),
        in_specs=[
            pl.BlockSpec((S, D), lambda p: (0, 0)),
            pl.BlockSpec((S, D), lambda p: (0, 0)),
            pl.BlockSpec((D, DC_SH), lambda p: (0, 0)),
            pl.BlockSpec((DC_SH, D), lambda p: (0, 0)),
            pl.BlockSpec((DC_SH, D), lambda p: (0, 0)),
            pl.BlockSpec((D, Dr), lambda p: (0, 0)),
            pl.BlockSpec((D, Dh), lambda p: (0, jnp.minimum(p, H - 1))),
            pl.BlockSpec((D, 2 * Dr), lambda p: (0, jnp.minimum(p, H - 1) // 2)),
        ],
        out_specs=pl.BlockSpec(
            (S, Dh), lambda p: (0, jnp.clip(p - H, 0, H - 1))),
        out_shape=jax.ShapeDtypeStruct((S, H * Dh), jnp.float32),
        scratch_shapes=[
            pltpu.VMEM((S, H * Dh), jnp.float32),
            pltpu.VMEM((H // 2, S, 2 * Dr), jnp.float32),
            pltpu.VMEM((S, Dr), jnp.float32),
            pltpu.VMEM((S, DC), jnp.float32),
            pltpu.VMEM((S, DC_SH), jnp.float32),
            pltpu.VMEM((DC, D), jnp.float32),
            pltpu.VMEM((DC, D), jnp.float32),
            pltpu.SemaphoreType.DMA((3,)),
            pltpu.SemaphoreType.DMA((3,)),
        ],
        compiler_params=pltpu.CompilerParams(collective_id=0),
    )(x_b, x_bp, wdkv, wuk, wuv, wkr, wq, wqr)


NCH = D // NBLK


def _cd_body(o_ref, wo_ref, out_ref, hs, sy, ry, sx, rx, s2, r2):
    n = pl.program_id(0)
    my_x = lax.axis_index("x")
    my_y = lax.axis_index("y")
    b_mine = 2 * my_x + my_y
    b_y = 2 * my_x + (1 - my_y)
    b_x = 2 * (1 - my_x) + my_y
    ypeer = (my_x, 1 - my_y)
    xpeer = (1 - my_x, my_y)

    def chunk(b, k):
        return out_ref.at[b, :, pl.ds(k * NBLK, NBLK)]

    @pl.when(n == 0)
    def _():
        barrier = pltpu.get_barrier_semaphore()
        for p in (ypeer, xpeer):
            pl.semaphore_signal(barrier, inc=1, device_id=p,
                                device_id_type=_MESH)
        pl.semaphore_wait(barrier, 2)
        pl.semaphore_signal(hs.at[0], inc=1, device_id=ypeer,
                            device_id_type=_MESH)
        pl.semaphore_signal(hs.at[1], inc=1, device_id=xpeer,
                            device_id_type=_MESH)
        pl.semaphore_wait(hs.at[0], 1)
        pl.semaphore_wait(hs.at[1], 1)

    @pl.when(n < NCH)
    def _():
        out_ref[b_mine, :, pl.ds(n * NBLK, NBLK)] = jnp.dot(
            o_ref[...], wo_ref[...], preferred_element_type=jnp.float32)
        pltpu.make_async_remote_copy(
            src_ref=chunk(b_mine, n), dst_ref=chunk(b_mine, n),
            send_sem=sy.at[n], recv_sem=ry.at[n],
            device_id=ypeer, device_id_type=_MESH,
        ).start()
        pltpu.make_async_remote_copy(
            src_ref=chunk(b_mine, n), dst_ref=chunk(b_mine, n),
            send_sem=sx.at[n], recv_sem=rx.at[n],
            device_id=xpeer, device_id_type=_MESH,
        ).start()

    @pl.when(n == NCH)
    def _():
        for k in range(NCH):
            pltpu.make_async_remote_copy(
                src_ref=chunk(b_mine, k), dst_ref=chunk(b_y, k),
                send_sem=sy.at[k], recv_sem=ry.at[k],
                device_id=ypeer, device_id_type=_MESH,
            ).wait_recv()
            pltpu.make_async_remote_copy(
                src_ref=chunk(b_mine, k), dst_ref=chunk(b_x, k),
                send_sem=sx.at[k], recv_sem=rx.at[k],
                device_id=xpeer, device_id_type=_MESH,
            ).wait_recv()
            pltpu.make_async_remote_copy(
                src_ref=chunk(b_mine, k), dst_ref=chunk(b_mine, k),
                send_sem=sy.at[k], recv_sem=ry.at[k],
                device_id=ypeer, device_id_type=_MESH,
            ).wait_send()
            pltpu.make_async_remote_copy(
                src_ref=chunk(b_mine, k), dst_ref=chunk(b_mine, k),
                send_sem=sx.at[k], recv_sem=rx.at[k],
                device_id=xpeer, device_id_type=_MESH,
            ).wait_send()

        r2y = pltpu.make_async_remote_copy(
            src_ref=out_ref.at[b_x, :, pl.ds(0, D // 2)],
            dst_ref=out_ref.at[b_x, :, pl.ds(0, D // 2)],
            send_sem=s2.at[0], recv_sem=r2.at[0],
            device_id=ypeer, device_id_type=_MESH,
        )
        r2x = pltpu.make_async_remote_copy(
            src_ref=out_ref.at[b_y, :, pl.ds(D // 2, D // 2)],
            dst_ref=out_ref.at[b_y, :, pl.ds(D // 2, D // 2)],
            send_sem=s2.at[1], recv_sem=r2.at[1],
            device_id=xpeer, device_id_type=_MESH,
        )
        r2y.start()
        r2x.start()
        r2y.wait()
        r2x.wait()


def _cd(o, wo):
    return pl.pallas_call(
        _cd_body,
        grid=(NCH + 1,),
        in_specs=[
            pl.BlockSpec((S, H * Dh), lambda n: (0, 0)),
            pl.BlockSpec((H * Dh, NBLK), lambda n: (0, jnp.minimum(n, NCH - 1))),
        ],
        out_specs=pl.BlockSpec((B, S, D), lambda n: (0, 0, 0)),
        out_shape=jax.ShapeDtypeStruct((B, S, D), jnp.float32),
        scratch_shapes=[
            pltpu.SemaphoreType.REGULAR((2,)),
            pltpu.SemaphoreType.DMA((NCH,)),
            pltpu.SemaphoreType.DMA((NCH,)),
            pltpu.SemaphoreType.DMA((NCH,)),
            pltpu.SemaphoreType.DMA((NCH,)),
            pltpu.SemaphoreType.DMA((2,)),
            pltpu.SemaphoreType.DMA((2,)),
        ],
        compiler_params=pltpu.CompilerParams(collective_id=1),
    )(o, wo)


def kernel(x, Wdkv, Wuk, Wuv, Wq, Wqr, Wkr, Wo):
    my_x = lax.axis_index("x")
    my_y = lax.axis_index("y")
    b_mine = 2 * my_x + my_y
    b_xpeer = 2 * (1 - my_x) + my_y
    x_b = lax.dynamic_slice_in_dim(x, b_mine, 1, axis=0)[0]
    x_bp = lax.dynamic_slice_in_dim(x, b_xpeer, 1, axis=0)[0]

    o = _ab(x_b, x_bp, Wdkv, Wuk, Wuv, Wkr, Wq, Wqr)
    return _cd(o, Wo)
